# Initial kernel scaffold; baseline (speedup 1.0000x reference)
import jax
import jax.numpy as jnp
from jax.experimental import pallas as pl


def kernel(fragment_idx, attach_table, attach_mask):
    raise NotImplementedError("write your pallas kernel here")


# speedup vs baseline: 2.7907x; 2.7907x over previous
import functools

import jax
import jax.numpy as jnp
from jax import lax
from jax.experimental import pallas as pl
from jax.experimental.pallas import tpu as pltpu
from jax.experimental.pallas import tpu_sc as plsc

NUM_FRAGMENTS = 128
MAX_ATTACH = 16
HIDDEN = 128
BATCH = 16384

NUM_CORES = 2
NUM_SUBCORES = 16
NUM_WORKERS = NUM_CORES * NUM_SUBCORES
BPW = BATCH // NUM_WORKERS
G = 16
BLK = MAX_ATTACH * HIDDEN


@jax.jit
def _fragment_gather(fragment_idx, attach_table, attach_mask):
  mesh = plsc.VectorSubcoreMesh(core_axis_name="c", subcore_axis_name="s")

  @functools.partial(
      pl.kernel,
      out_type=(
          jax.ShapeDtypeStruct((BATCH * MAX_ATTACH * HIDDEN,), jnp.float32),
          jax.ShapeDtypeStruct((BATCH, MAX_ATTACH), jnp.float32),
      ),
      mesh=mesh,
      scratch_types=[
          pltpu.VMEM((BPW,), jnp.int32),
          pltpu.VMEM((G, MAX_ATTACH * HIDDEN), jnp.float32),
          pltpu.VMEM((NUM_FRAGMENTS, MAX_ATTACH), jnp.float32),
          pltpu.VMEM((BPW, MAX_ATTACH), jnp.float32),
          pltpu.SemaphoreType.DMA,
          pltpu.SemaphoreType.DMA,
      ],
  )
  def k(fi_hbm, tab_hbm, msk_hbm, oemb, omsk, fi_v, buf, mvmem, mout, insem,
        outsem):
    wid = lax.axis_index("s") * NUM_CORES + lax.axis_index("c")
    base = wid * BPW
    pltpu.sync_copy(fi_hbm.at[pl.ds(base, BPW)], fi_v)
    pltpu.sync_copy(msk_hbm, mvmem)

    @pl.loop(0, BPW, step=G)
    def _(b0):
      fvec = fi_v[pl.ds(b0, G)]
      in_copies = []
      for t in range(G):
        src0 = fvec[t] * HIDDEN
        in_copies.append(
            pltpu.async_copy(tab_hbm.at[pl.ds(src0, BLK)], buf.at[t], insem))
      for t in range(G):
        mout[b0 + t] = mvmem[fvec[t]]
      out_copies = []
      for t in range(G):
        in_copies[t].wait()
        dst0 = (base + b0 + t) * BLK
        out_copies.append(
            pltpu.async_copy(buf.at[t], oemb.at[pl.ds(dst0, BLK)], outsem))
      for t in range(G):
        out_copies[t].wait()

    pltpu.sync_copy(mout, omsk.at[pl.ds(base, BPW)])

  return k(fragment_idx, attach_table.reshape(-1), attach_mask)


def kernel(fragment_idx, attach_table, attach_mask):
  fi = fragment_idx
  if fi.ndim == 0:
    fi = fi[None]
  fi = fi.astype(jnp.int32)
  emb_flat, mask_out = _fragment_gather(fi, attach_table, attach_mask)
  emb = emb_flat.reshape(BATCH, MAX_ATTACH, HIDDEN)
  return emb, mask_out


# speedup vs baseline: 2.8550x; 1.0230x over previous
import functools

import jax
import jax.numpy as jnp
from jax import lax
from jax.experimental import pallas as pl
from jax.experimental.pallas import tpu as pltpu
from jax.experimental.pallas import tpu_sc as plsc

NUM_FRAGMENTS = 128
MAX_ATTACH = 16
HIDDEN = 128
BATCH = 16384

NUM_CORES = 2
NUM_SUBCORES = 16
NUM_WORKERS = NUM_CORES * NUM_SUBCORES
BPW = BATCH // NUM_WORKERS
G = 16
BLK = MAX_ATTACH * HIDDEN
GBLK = G * BLK


@jax.jit
def _fragment_gather(fragment_idx, attach_table, attach_mask):
  mesh = plsc.VectorSubcoreMesh(core_axis_name="c", subcore_axis_name="s")

  @functools.partial(
      pl.kernel,
      out_type=(
          jax.ShapeDtypeStruct((BATCH * MAX_ATTACH * HIDDEN,), jnp.float32),
          jax.ShapeDtypeStruct((BATCH * MAX_ATTACH,), jnp.float32),
      ),
      mesh=mesh,
      scratch_types=[
          pltpu.VMEM((BPW,), jnp.int32),
          pltpu.VMEM((GBLK,), jnp.float32),
          pltpu.VMEM((GBLK,), jnp.float32),
          pltpu.VMEM((NUM_FRAGMENTS * MAX_ATTACH,), jnp.float32),
          pltpu.VMEM((BPW * MAX_ATTACH,), jnp.float32),
          pltpu.SemaphoreType.DMA,
          pltpu.SemaphoreType.DMA,
          pltpu.SemaphoreType.DMA,
          pltpu.SemaphoreType.DMA,
      ],
  )
  def k(fi_hbm, tab_hbm, msk_hbm, oemb, omsk, fi_v, buf0, buf1, mvmem, mout,
        insem0, insem1, outsem0, outsem1):
    wid = lax.axis_index("s") * NUM_CORES + lax.axis_index("c")
    base = wid * BPW
    pltpu.sync_copy(fi_hbm.at[pl.ds(base, BPW)], fi_v)
    pltpu.sync_copy(msk_hbm, mvmem)

    def fire_in(buf, sem, b0):
      fvec = fi_v[pl.ds(b0, G)]
      for t in range(G):
        src0 = fvec[t] * HIDDEN
        pltpu.async_copy(tab_hbm.at[pl.ds(src0, BLK)],
                         buf.at[pl.ds(t * BLK, BLK)], sem)
      for t in range(G):
        mout[pl.ds((b0 + t) * MAX_ATTACH, MAX_ATTACH)] = (
            mvmem[pl.ds(fvec[t] * MAX_ATTACH, MAX_ATTACH)])

    def drain_in(buf, sem):
      pltpu.make_async_copy(tab_hbm.at[pl.ds(0, GBLK)], buf, sem).wait()

    def fire_out(buf, sem, b0):
      pltpu.async_copy(buf, oemb.at[pl.ds((base + b0) * BLK, GBLK)], sem)

    def drain_out(buf, sem):
      pltpu.make_async_copy(buf, oemb.at[pl.ds(0, GBLK)], sem).wait()

    fire_in(buf0, insem0, 0)

    @pl.loop(0, BPW, step=2 * G)
    def _(b0):
      @pl.when(b0 > 0)
      def _():
        drain_out(buf1, outsem1)

      fire_in(buf1, insem1, b0 + G)
      drain_in(buf0, insem0)
      fire_out(buf0, outsem0, b0)
      drain_in(buf1, insem1)
      drain_out(buf0, outsem0)

      @pl.when(b0 + 2 * G < BPW)
      def _():
        fire_in(buf0, insem0, b0 + 2 * G)

      fire_out(buf1, outsem1, b0 + G)

    drain_out(buf1, outsem1)

    pltpu.sync_copy(mout, omsk.at[pl.ds(base * MAX_ATTACH, BPW * MAX_ATTACH)])

  return k(fragment_idx, attach_table.reshape(-1), attach_mask.reshape(-1))


def kernel(fragment_idx, attach_table, attach_mask):
  fi = fragment_idx
  if fi.ndim == 0:
    fi = fi[None]
  fi = fi.astype(jnp.int32)
  emb_flat, mask_flat = _fragment_gather(fi, attach_table, attach_mask)
  emb = emb_flat.reshape(BATCH, MAX_ATTACH, HIDDEN)
  return emb, mask_flat.reshape(BATCH, MAX_ATTACH)


# speedup vs baseline: 9.8909x; 3.4644x over previous
import functools

import jax
import jax.numpy as jnp
from jax import lax
from jax.experimental import pallas as pl
from jax.experimental.pallas import tpu as pltpu
from jax.experimental.pallas import tpu_sc as plsc

NUM_FRAGMENTS = 128
MAX_ATTACH = 16
HIDDEN = 128
BATCH = 16384

NUM_CORES = 2
NUM_SUBCORES = 16
NUM_WORKERS = NUM_CORES * NUM_SUBCORES
BPW = BATCH // NUM_WORKERS
G = 16
BLK = MAX_ATTACH * HIDDEN
GBLK = G * BLK
TABW = (NUM_FRAGMENTS + MAX_ATTACH) * HIDDEN


@jax.jit
def _fragment_gather(fragment_idx, attach_table, attach_mask):
  mesh = plsc.VectorSubcoreMesh(core_axis_name="c", subcore_axis_name="s")

  @functools.partial(
      pl.kernel,
      out_type=(
          jax.ShapeDtypeStruct((BATCH * MAX_ATTACH * HIDDEN,), jnp.float32),
          jax.ShapeDtypeStruct((BATCH * MAX_ATTACH,), jnp.float32),
      ),
      mesh=mesh,
      scratch_types=[
          pltpu.VMEM_SHARED((NUM_FRAGMENTS * MAX_ATTACH * HIDDEN,),
                            jnp.float32),
          pltpu.VMEM((BPW,), jnp.int32),
          pltpu.VMEM((NUM_FRAGMENTS * MAX_ATTACH,), jnp.float32),
          pltpu.VMEM((BPW * MAX_ATTACH,), jnp.float32),
          pltpu.SemaphoreType.DMA,
          pltpu.SemaphoreType.DMA,
      ],
  )
  def k(fi_hbm, tab_hbm, msk_hbm, oemb, omsk, stab, fi_v, mvmem, mout, outsem,
        auxsem):
    wid = lax.axis_index("s") * NUM_CORES + lax.axis_index("c")
    base = wid * BPW
    pltpu.sync_copy(fi_hbm.at[pl.ds(base, BPW)], fi_v)
    pltpu.sync_copy(msk_hbm, mvmem)

    @pl.when(lax.axis_index("s") == 0)
    def _():
      pltpu.sync_copy(tab_hbm, stab)

    plsc.subcore_barrier()

    def fire_group(b0):
      fvec = fi_v[pl.ds(b0, G)]
      for t in range(G):
        src0 = fvec[t] * HIDDEN
        dst0 = (base + b0 + t) * BLK
        pltpu.async_copy(stab.at[pl.ds(src0, BLK)], oemb.at[pl.ds(dst0, BLK)],
                         outsem)
      for t in range(G):
        mout[pl.ds((b0 + t) * MAX_ATTACH, MAX_ATTACH)] = (
            mvmem[pl.ds(fvec[t] * MAX_ATTACH, MAX_ATTACH)])

    def drain_group():
      pltpu.make_async_copy(tab_hbm.at[pl.ds(0, GBLK)],
                            oemb.at[pl.ds(0, GBLK)], outsem).wait()

    fire_group(0)

    @pl.loop(G, BPW, step=G)
    def _(b0):
      fire_group(b0)
      drain_group()

    drain_group()

    pltpu.async_copy(mout, omsk.at[pl.ds(base * MAX_ATTACH, BPW * MAX_ATTACH)],
                     auxsem).wait()

  return k(fragment_idx, attach_table.reshape(-1), attach_mask.reshape(-1))


def kernel(fragment_idx, attach_table, attach_mask):
  fi = fragment_idx
  if fi.ndim == 0:
    fi = fi[None]
  fi = fi.astype(jnp.int32)
  emb_flat, mask_flat = _fragment_gather(fi, attach_table, attach_mask)
  emb = emb_flat.reshape(BATCH, MAX_ATTACH, HIDDEN)
  return emb, mask_flat.reshape(BATCH, MAX_ATTACH)
